# baseline (speedup 1.0000x reference)
import jax
import jax.numpy as jnp
from jax import lax
from jax.experimental import pallas as pl
from jax.experimental.pallas import tpu as pltpu
from jax.experimental.pallas import tpu_sc as plsc

E = 65536
N = 4096
V = 8192
NC = 2
NS = 16
NW = NC * NS
PW = (N * V) // NW
RW = N // NW
CH = 16384
NCH = PW // CH
RPC = CH // V
DELW = E // NW
L = 16
SUP = 8
NSUP = NCH // SUP


def _sc_body(act_ref, d_ref, ad_ref, arm_ref, out_ref,
             actv, adv, scales, inb0, inb1, outb0, outb1,
             sin0, sin1, sout0, sout1):
    wid = lax.axis_index("s") * NC + lax.axis_index("c")

    pltpu.sync_copy(act_ref, actv)
    for s in range(NSUP):
        pltpu.sync_copy(
            ad_ref.at[pl.ds(RPC * SUP * (s * NW + wid), RPC * SUP)],
            adv.at[pl.ds(RPC * SUP * s, RPC * SUP)])
    av = actv[...]
    a0v = jnp.full((L,), av[0], jnp.float32)
    a1 = av[1]

    for j in range(RW // L):
        vec = adv[pl.ds(j * L, L)] * a1
        for i in range(L):
            scales[pl.ds((j * L + i) * L, L)] = jnp.full((L,), vec[i],
                                                         jnp.float32)

    inb = (inb0, inb1)
    outb = (outb0, outb1)
    sin = (sin0, sin1)
    sout = (sout0, sout1)

    def gchunk(t):
        return ((t // SUP) * NW + wid) * SUP + (t % SUP)

    pltpu.async_copy(arm_ref.at[pl.ds(gchunk(0) * CH, CH)], inb0, sin0)

    pltpu.sync_copy(d_ref.at[pl.ds(wid * DELW, DELW)], inb1.at[pl.ds(0, DELW)])

    @plsc.parallel_loop(0, DELW, step=L, unroll=16)
    def _(i):
        outb1[pl.ds(i, L)] = inb1[pl.ds(i, L)] * a0v

    pltpu.sync_copy(outb1.at[pl.ds(0, DELW)],
                    out_ref.at[pl.ds(wid * DELW, DELW)])

    @pl.loop(0, NCH, step=2)
    def _(t0):
        for b in range(2):
            t = t0 + b
            nb = 1 - b

            @pl.when(t + 1 < NCH)
            def _():
                pltpu.async_copy(
                    arm_ref.at[pl.ds(gchunk(t + 1) * CH, CH)],
                    inb[nb], sin[nb])

            pltpu.make_async_copy(
                arm_ref.at[pl.ds(0, CH)], inb[b], sin[b]).wait()

            @pl.when(t >= 2)
            def _():
                pltpu.make_async_copy(
                    outb[b], out_ref.at[pl.ds(0, CH)], sout[b]).wait()

            for r in range(RPC):
                srow = scales[pl.ds((t * RPC + r) * L, L)]

                @plsc.parallel_loop(r * V, (r + 1) * V, step=L, unroll=16)
                def _(i):
                    outb[b][pl.ds(i, L)] = inb[b][pl.ds(i, L)] * srow

            pltpu.async_copy(
                outb[b], out_ref.at[pl.ds(E + gchunk(t) * CH, CH)], sout[b])

    pltpu.make_async_copy(outb0, out_ref.at[pl.ds(0, CH)], sout0).wait()
    pltpu.make_async_copy(outb1, out_ref.at[pl.ds(0, CH)], sout1).wait()


def kernel(act_prob, idx_del_prob, idx_add_prob, idx_arm_prob):
    act16 = jnp.pad(act_prob.reshape(-1), (0, 14))
    d = idx_del_prob.reshape(-1)
    ad = idx_add_prob.reshape(-1)
    arm = idx_arm_prob.reshape(-1)

    mesh = plsc.VectorSubcoreMesh(
        core_axis_name="c", subcore_axis_name="s",
        num_cores=NC, num_subcores=NS)
    f = pl.kernel(
        _sc_body,
        out_type=jax.ShapeDtypeStruct((E + N * V,), jnp.float32),
        mesh=mesh,
        scratch_types=[
            pltpu.VMEM((L,), jnp.float32),
            pltpu.VMEM((RW,), jnp.float32),
            pltpu.VMEM((RW * L,), jnp.float32),
            pltpu.VMEM((CH,), jnp.float32),
            pltpu.VMEM((CH,), jnp.float32),
            pltpu.VMEM((CH,), jnp.float32),
            pltpu.VMEM((CH,), jnp.float32),
            pltpu.SemaphoreType.DMA,
            pltpu.SemaphoreType.DMA,
            pltpu.SemaphoreType.DMA,
            pltpu.SemaphoreType.DMA,
        ],
    )
    return f(act16, d, ad, arm)


# speedup vs baseline: 1.0072x; 1.0072x over previous
import jax
import jax.numpy as jnp
from jax import lax
from jax.experimental import pallas as pl
from jax.experimental.pallas import tpu as pltpu
from jax.experimental.pallas import tpu_sc as plsc

E = 65536
N = 4096
V = 8192
NC = 2
NS = 16
NW = NC * NS
PW = (N * V) // NW
RW = N // NW
CH = 16384
NCH = PW // CH
RPC = CH // V
DELW = E // NW
L = 16


def _sc_body(act_ref, d_ref, ad_ref, arm_ref, out_ref,
             actv, adv, scales, inb0, inb1, outb0, outb1,
             sin0, sin1, sout0, sout1):
    wid = lax.axis_index("s") * NC + lax.axis_index("c")
    arm_base = wid * PW
    out_base = E + arm_base

    pltpu.sync_copy(act_ref, actv)
    pltpu.sync_copy(ad_ref.at[pl.ds(wid * RW, RW)], adv)
    av = actv[...]
    a0v = jnp.full((L,), av[0], jnp.float32)
    a1 = av[1]

    for j in range(RW // L):
        vec = adv[pl.ds(j * L, L)] * a1
        for i in range(L):
            scales[pl.ds((j * L + i) * L, L)] = jnp.full((L,), vec[i],
                                                         jnp.float32)

    inb = (inb0, inb1)
    outb = (outb0, outb1)
    sin = (sin0, sin1)
    sout = (sout0, sout1)

    pltpu.async_copy(arm_ref.at[pl.ds(arm_base, CH)], inb0, sin0)

    pltpu.sync_copy(d_ref.at[pl.ds(wid * DELW, DELW)], inb1.at[pl.ds(0, DELW)])

    @plsc.parallel_loop(0, DELW, step=L, unroll=16)
    def _(i):
        outb1[pl.ds(i, L)] = inb1[pl.ds(i, L)] * a0v

    pltpu.sync_copy(outb1.at[pl.ds(0, DELW)],
                    out_ref.at[pl.ds(wid * DELW, DELW)])

    @pl.loop(0, NCH, step=2)
    def _(t0):
        for b in range(2):
            t = t0 + b
            nb = 1 - b

            @pl.when(t + 1 < NCH)
            def _():
                pltpu.async_copy(
                    arm_ref.at[pl.ds(arm_base + (t + 1) * CH, CH)],
                    inb[nb], sin[nb])

            pltpu.make_async_copy(
                arm_ref.at[pl.ds(arm_base, CH)], inb[b], sin[b]).wait()

            @pl.when(t >= 2)
            def _():
                pltpu.make_async_copy(
                    outb[b], out_ref.at[pl.ds(out_base, CH)], sout[b]).wait()

            for r in range(RPC):
                srow = scales[pl.ds((t * RPC + r) * L, L)]

                @plsc.parallel_loop(r * V, (r + 1) * V, step=L, unroll=16)
                def _(i):
                    outb[b][pl.ds(i, L)] = inb[b][pl.ds(i, L)] * srow

            pltpu.async_copy(
                outb[b], out_ref.at[pl.ds(out_base + t * CH, CH)], sout[b])

    pltpu.make_async_copy(outb0, out_ref.at[pl.ds(out_base, CH)], sout0).wait()
    pltpu.make_async_copy(outb1, out_ref.at[pl.ds(out_base, CH)], sout1).wait()


def kernel(act_prob, idx_del_prob, idx_add_prob, idx_arm_prob):
    act16 = jnp.pad(act_prob.reshape(-1), (0, 14))
    d = idx_del_prob.reshape(-1)
    ad = idx_add_prob.reshape(-1)
    arm = idx_arm_prob.reshape(-1)

    mesh = plsc.VectorSubcoreMesh(
        core_axis_name="c", subcore_axis_name="s",
        num_cores=NC, num_subcores=NS)
    f = pl.kernel(
        _sc_body,
        out_type=jax.ShapeDtypeStruct((E + N * V,), jnp.float32),
        mesh=mesh,
        scratch_types=[
            pltpu.VMEM((L,), jnp.float32),
            pltpu.VMEM((RW,), jnp.float32),
            pltpu.VMEM((RW * L,), jnp.float32),
            pltpu.VMEM((CH,), jnp.float32),
            pltpu.VMEM((CH,), jnp.float32),
            pltpu.VMEM((CH,), jnp.float32),
            pltpu.VMEM((CH,), jnp.float32),
            pltpu.SemaphoreType.DMA,
            pltpu.SemaphoreType.DMA,
            pltpu.SemaphoreType.DMA,
            pltpu.SemaphoreType.DMA,
        ],
    )
    return f(act16, d, ad, arm)
